# baseline (speedup 1.0000x reference)
import functools

import jax
import jax.numpy as jnp
from jax import lax
from jax.experimental import pallas as pl
from jax.experimental.pallas import tpu as pltpu
from jax.experimental.pallas import tpu_sc as plsc

NC = 2
NS = 16
L = 16
NW = NC * NS
ROWS = 128
COLS = 32768
N = ROWS * COLS
SLAB = N // NW
CH = 8192
NCHUNK = SLAB // CH
VPC = CH // L
BIG = 2**31 - 1
NEG = float("-inf")

_MESH = plsc.VectorSubcoreMesh(core_axis_name="c", subcore_axis_name="s")


_GDN = lax.GatherDimensionNumbers(
    offset_dims=(), collapsed_slice_dims=(0,), start_index_map=(0,))


def _shuffle(v, idx):
    return lax.gather(v, idx[:, None], _GDN, (1,),
                      mode=lax.GatherScatterMode.PROMISE_IN_BOUNDS)


def _lane_reduce(v, op, lane):
    for sh in (8, 4, 2, 1):
        v = op(v, _shuffle(v, lane ^ sh))
    return v


@functools.partial(
    pl.kernel,
    out_type=(
        jax.ShapeDtypeStruct((NW, L), jnp.float32),
        jax.ShapeDtypeStruct((NW, L), jnp.int32),
    ),
    mesh=_MESH,
    scratch_types=[
        pltpu.VMEM((CH,), jnp.float32),
        pltpu.VMEM((CH,), jnp.float32),
        pltpu.VMEM((L,), jnp.float32),
        pltpu.VMEM((L,), jnp.int32),
        pltpu.SemaphoreType.DMA,
        pltpu.SemaphoreType.DMA,
    ],
)
def _sc_scan(x_hbm, vals_hbm, idxs_hbm, buf0, buf1, stage_v, stage_i,
             sem0, sem1):
    wid = lax.axis_index("s") * NC + lax.axis_index("c")
    base = wid * SLAB
    bufs = (buf0, buf1)
    sems = (sem0, sem1)
    copies = [None, None]

    def start(c):
        b = c % 2
        off = pl.multiple_of(base + c * CH, CH)
        copies[b] = pltpu.async_copy(x_hbm.at[pl.ds(off, CH)], bufs[b],
                                     sems[b])

    start(0)
    lane = lax.iota(jnp.int32, L)
    cmv = jnp.full((L,), NEG, jnp.float32)
    for c in range(NCHUNK):
        if c + 1 < NCHUNK:
            start(c + 1)
        b = c % 2
        copies[b].wait()
        buf = bufs[b]

        def body(i, accs, buf=buf):
            a0, a1, a2, a3 = accs
            o = i * (4 * L)
            return (
                jnp.maximum(a0, buf[pl.ds(o, L)]),
                jnp.maximum(a1, buf[pl.ds(o + L, L)]),
                jnp.maximum(a2, buf[pl.ds(o + 2 * L, L)]),
                jnp.maximum(a3, buf[pl.ds(o + 3 * L, L)]),
            )

        init = tuple(jnp.full((L,), NEG, jnp.float32) for _ in range(4))
        a0, a1, a2, a3 = lax.fori_loop(0, VPC // 4, body, init)
        acc = jnp.maximum(jnp.maximum(a0, a1), jnp.maximum(a2, a3))
        m_c = _lane_reduce(acc, jnp.maximum, lane)
        cmv = jnp.where(lane == c, m_c, cmv)

    wmax = _lane_reduce(cmv, jnp.maximum, lane)
    cid_v = _lane_reduce(jnp.where(cmv == wmax, lane, BIG), jnp.minimum,
                         lane)
    cid = cid_v[0]

    off = pl.multiple_of(base + cid * CH, CH)
    pltpu.async_copy(x_hbm.at[pl.ds(off, CH)], buf0, sem0).wait()

    def rbody(i, ridx):
        v = buf0[pl.ds(i * L, L)]
        upd = (v == wmax) & (ridx == BIG)
        return jnp.where(upd, i, ridx)

    ridx = lax.fori_loop(0, VPC, rbody, jnp.full((L,), BIG, jnp.int32))
    flat = base + cid * CH + ridx * L + lane
    flat = jnp.where(ridx == BIG, BIG, flat)
    widx = _lane_reduce(flat, jnp.minimum, lane)

    stage_v[...] = wmax
    stage_i[...] = widx
    pltpu.sync_copy(stage_v, vals_hbm.at[wid])
    pltpu.sync_copy(stage_i, idxs_hbm.at[wid])


def _merge_body(vals_ref, idxs_ref, out_ref):
    def body(w, carry):
        bv, bi = carry
        v = vals_ref[w, 0]
        ix = idxs_ref[w, 0]
        better = (v > bv) | ((v == bv) & (ix < bi))
        return (jnp.where(better, v, bv), jnp.where(better, ix, bi))

    _, bi = lax.fori_loop(0, NW, body,
                          (jnp.float32(NEG), jnp.int32(BIG)))
    out_ref[0] = bi


def kernel(x):
    vals, idxs = _sc_scan(x.reshape(-1))
    merged = pl.pallas_call(
        _merge_body,
        in_specs=[
            pl.BlockSpec(memory_space=pltpu.SMEM),
            pl.BlockSpec(memory_space=pltpu.SMEM),
        ],
        out_specs=pl.BlockSpec(memory_space=pltpu.SMEM),
        out_shape=jax.ShapeDtypeStruct((1,), jnp.int32),
    )(vals, idxs)
    return merged[0].astype(jnp.int64)


# speedup vs baseline: 1.4611x; 1.4611x over previous
import functools

import jax
import jax.numpy as jnp
from jax import lax
from jax.experimental import pallas as pl
from jax.experimental.pallas import tpu as pltpu
from jax.experimental.pallas import tpu_sc as plsc

NC = 2
NS = 16
L = 16
NW = NC * NS
ROWS = 128
COLS = 32768
RPW = ROWS // NW
CH = 8192
CPR = COLS // CH
NCHUNK = RPW * CPR
VPC = CH // L
UNROLL = 8
BIG = 2**31 - 1
NEG = float("-inf")

_MESH = plsc.VectorSubcoreMesh(core_axis_name="c", subcore_axis_name="s")

_GDN = lax.GatherDimensionNumbers(
    offset_dims=(), collapsed_slice_dims=(0,), start_index_map=(0,))


def _shuffle(v, idx):
    return lax.gather(v, idx[:, None], _GDN, (1,),
                      mode=lax.GatherScatterMode.PROMISE_IN_BOUNDS)


def _lane_reduce(v, op, lane):
    for sh in (8, 4, 2, 1):
        v = op(v, _shuffle(v, lane ^ sh))
    return v


@functools.partial(
    pl.kernel,
    out_type=(
        jax.ShapeDtypeStruct((NW, L), jnp.float32),
        jax.ShapeDtypeStruct((NW, L), jnp.int32),
    ),
    mesh=_MESH,
    scratch_types=[
        pltpu.VMEM((CH,), jnp.float32),
        pltpu.VMEM((CH,), jnp.float32),
        pltpu.VMEM((L,), jnp.float32),
        pltpu.VMEM((L,), jnp.int32),
        pltpu.SemaphoreType.DMA,
        pltpu.SemaphoreType.DMA,
    ],
)
def _sc_scan(x_hbm, vals_hbm, idxs_hbm, buf0, buf1, stage_v, stage_i,
             sem0, sem1):
    wid = lax.axis_index("s") * NC + lax.axis_index("c")
    row0 = wid * RPW
    bufs = (buf0, buf1)
    sems = (sem0, sem1)
    copies = [None, None]

    def start(c):
        b = c % 2
        row = row0 + c // CPR
        col = pl.multiple_of((c % CPR) * CH, CH)
        copies[b] = pltpu.async_copy(x_hbm.at[row, pl.ds(col, CH)],
                                     bufs[b], sems[b])

    start(0)
    lane = lax.iota(jnp.int32, L)
    cmv = jnp.full((L,), NEG, jnp.float32)
    for c in range(NCHUNK):
        if c + 1 < NCHUNK:
            start(c + 1)
        b = c % 2
        copies[b].wait()
        buf = bufs[b]

        def body(i, accs, buf=buf):
            o = i * (UNROLL * L)
            return tuple(
                jnp.maximum(accs[u], buf[pl.ds(o + u * L, L)])
                for u in range(UNROLL)
            )

        init = tuple(jnp.full((L,), NEG, jnp.float32)
                     for _ in range(UNROLL))
        accs = lax.fori_loop(0, VPC // UNROLL, body, init)
        acc = accs[0]
        for u in range(1, UNROLL):
            acc = jnp.maximum(acc, accs[u])
        m_c = _lane_reduce(acc, jnp.maximum, lane)
        cmv = jnp.where(lane == c, m_c, cmv)

    wmax = _lane_reduce(cmv, jnp.maximum, lane)
    cid_v = _lane_reduce(jnp.where(cmv == wmax, lane, BIG), jnp.minimum,
                         lane)
    cid = cid_v[0]

    rrow = row0 + cid // CPR
    rcol = pl.multiple_of((cid % CPR) * CH, CH)
    pltpu.async_copy(x_hbm.at[rrow, pl.ds(rcol, CH)], buf0, sem0).wait()

    def rbody(i, ridx):
        v = buf0[pl.ds(i * L, L)]
        upd = (v == wmax) & (ridx == BIG)
        return jnp.where(upd, i, ridx)

    ridx = lax.fori_loop(0, VPC, rbody, jnp.full((L,), BIG, jnp.int32))
    flat = rrow * COLS + rcol + ridx * L + lane
    flat = jnp.where(ridx == BIG, BIG, flat)
    widx = _lane_reduce(flat, jnp.minimum, lane)

    stage_v[...] = wmax
    stage_i[...] = widx
    pltpu.sync_copy(stage_v, vals_hbm.at[wid])
    pltpu.sync_copy(stage_i, idxs_hbm.at[wid])


def _merge_body(vals_ref, idxs_ref, out_ref):
    def body(w, carry):
        bv, bi = carry
        v = vals_ref[w, 0]
        ix = idxs_ref[w, 0]
        better = (v > bv) | ((v == bv) & (ix < bi))
        return (jnp.where(better, v, bv), jnp.where(better, ix, bi))

    _, bi = lax.fori_loop(0, NW, body,
                          (jnp.float32(NEG), jnp.int32(BIG)))
    out_ref[0] = bi


def kernel(x):
    vals, idxs = _sc_scan(x)
    merged = pl.pallas_call(
        _merge_body,
        in_specs=[
            pl.BlockSpec(memory_space=pltpu.SMEM),
            pl.BlockSpec(memory_space=pltpu.SMEM),
        ],
        out_specs=pl.BlockSpec(memory_space=pltpu.SMEM),
        out_shape=jax.ShapeDtypeStruct((1,), jnp.int32),
    )(vals, idxs)
    return merged[0].astype(jnp.int64)


# speedup vs baseline: 1.5430x; 1.0560x over previous
import functools

import jax
import jax.numpy as jnp
from jax import lax
from jax.experimental import pallas as pl
from jax.experimental.pallas import tpu as pltpu
from jax.experimental.pallas import tpu_sc as plsc

NC = 2
NS = 16
L = 16
NW = NC * NS
ROWS = 128
COLS = 32768
RPW = ROWS // NW
CH = 16384
CPR = COLS // CH
NCHUNK = RPW * CPR
VPC = CH // L
UNROLL = 8
BIG = 2**31 - 1
NEG = float("-inf")

_MESH = plsc.VectorSubcoreMesh(core_axis_name="c", subcore_axis_name="s")

_GDN = lax.GatherDimensionNumbers(
    offset_dims=(), collapsed_slice_dims=(0,), start_index_map=(0,))


def _shuffle(v, idx):
    return lax.gather(v, idx[:, None], _GDN, (1,),
                      mode=lax.GatherScatterMode.PROMISE_IN_BOUNDS)


def _lane_reduce(v, op, lane):
    for sh in (8, 4, 2, 1):
        v = op(v, _shuffle(v, lane ^ sh))
    return v


@functools.partial(
    pl.kernel,
    out_type=(
        jax.ShapeDtypeStruct((NW, L), jnp.float32),
        jax.ShapeDtypeStruct((NW, L), jnp.int32),
    ),
    mesh=_MESH,
    scratch_types=[
        pltpu.VMEM((CH,), jnp.float32),
        pltpu.VMEM((CH,), jnp.float32),
        pltpu.VMEM((L,), jnp.float32),
        pltpu.VMEM((L,), jnp.int32),
        pltpu.SemaphoreType.DMA,
        pltpu.SemaphoreType.DMA,
    ],
)
def _sc_scan(x_hbm, vals_hbm, idxs_hbm, buf0, buf1, stage_v, stage_i,
             sem0, sem1):
    wid = lax.axis_index("s") * NC + lax.axis_index("c")
    row0 = wid * RPW
    bufs = (buf0, buf1)
    sems = (sem0, sem1)
    lane = lax.iota(jnp.int32, L)

    def start_dyn(c, b):
        row = row0 + c // CPR
        col = pl.multiple_of((c % CPR) * CH, CH)
        pltpu.async_copy(x_hbm.at[row, pl.ds(col, CH)], bufs[b], sems[b])

    def scan_chunk(buf):
        init = tuple(jnp.full((L,), NEG, jnp.float32)
                     for _ in range(UNROLL))

        @plsc.parallel_loop(0, VPC, step=UNROLL, unroll=2, carry=init)
        def accs(i, accs_in):
            return tuple(
                jnp.maximum(accs_in[u], buf[pl.ds((i + u) * L, L)])
                for u in range(UNROLL)
            )

        acc = accs[0]
        for u in range(1, UNROLL):
            acc = jnp.maximum(acc, accs[u])
        return acc

    def chunk_step(c, b, cmv):
        @pl.when(c + 2 < NCHUNK)
        def _():
            start_dyn(c + 2, b)
        pltpu.make_async_copy(x_hbm.at[0, pl.ds(0, CH)], bufs[b],
                              sems[b]).wait()
        m_c = _lane_reduce(scan_chunk(bufs[b]), jnp.maximum, lane)
        return jnp.where(lane == c, m_c, cmv)

    start_dyn(0, 0)
    start_dyn(1, 1)

    @pl.loop(0, NCHUNK, step=2,
             init_carry=jnp.full((L,), NEG, jnp.float32))
    def cmv(c2, cmv_in):
        cmv_in = chunk_step(c2, 0, cmv_in)
        return chunk_step(c2 + 1, 1, cmv_in)

    wmax = _lane_reduce(cmv, jnp.maximum, lane)
    cid_v = _lane_reduce(jnp.where(cmv == wmax, lane, BIG), jnp.minimum,
                         lane)
    cid = cid_v[0]

    rrow = row0 + cid // CPR
    rcol = pl.multiple_of((cid % CPR) * CH, CH)
    pltpu.async_copy(x_hbm.at[rrow, pl.ds(rcol, CH)], buf0, sem0).wait()

    def rbody(i, ridx):
        v = buf0[pl.ds(i * L, L)]
        upd = (v == wmax) & (ridx == BIG)
        return jnp.where(upd, i, ridx)

    ridx = lax.fori_loop(0, VPC, rbody, jnp.full((L,), BIG, jnp.int32))
    flat = rrow * COLS + rcol + ridx * L + lane
    flat = jnp.where(ridx == BIG, BIG, flat)
    widx = _lane_reduce(flat, jnp.minimum, lane)

    stage_v[...] = wmax
    stage_i[...] = widx
    pltpu.sync_copy(stage_v, vals_hbm.at[wid])
    pltpu.sync_copy(stage_i, idxs_hbm.at[wid])


def _merge_body(vals_ref, idxs_ref, out_ref):
    def body(w, carry):
        bv, bi = carry
        v = vals_ref[w, 0]
        ix = idxs_ref[w, 0]
        better = (v > bv) | ((v == bv) & (ix < bi))
        return (jnp.where(better, v, bv), jnp.where(better, ix, bi))

    _, bi = lax.fori_loop(0, NW, body,
                          (jnp.float32(NEG), jnp.int32(BIG)))
    out_ref[0] = bi


def kernel(x):
    vals, idxs = _sc_scan(x)
    merged = pl.pallas_call(
        _merge_body,
        in_specs=[
            pl.BlockSpec(memory_space=pltpu.SMEM),
            pl.BlockSpec(memory_space=pltpu.SMEM),
        ],
        out_specs=pl.BlockSpec(memory_space=pltpu.SMEM),
        out_shape=jax.ShapeDtypeStruct((1,), jnp.int32),
    )(vals, idxs)
    return merged[0].astype(jnp.int64)


# speedup vs baseline: 1.6806x; 1.0892x over previous
import functools

import jax
import jax.numpy as jnp
from jax import lax
from jax.experimental import pallas as pl
from jax.experimental.pallas import tpu as pltpu
from jax.experimental.pallas import tpu_sc as plsc

NC = 2
NS = 16
L = 16
NW = NC * NS
ROWS = 128
COLS = 32768
RPW = ROWS // NW
CH = 16384
CPR = COLS // CH
NCHUNK = RPW * CPR
VPC = CH // L
UNROLL = 8
NBUF = 4
RU = 4
BIG = 2**31 - 1
NEG = float("-inf")

_MESH = plsc.VectorSubcoreMesh(core_axis_name="c", subcore_axis_name="s")

_GDN = lax.GatherDimensionNumbers(
    offset_dims=(), collapsed_slice_dims=(0,), start_index_map=(0,))


def _shuffle(v, idx):
    return lax.gather(v, idx[:, None], _GDN, (1,),
                      mode=lax.GatherScatterMode.PROMISE_IN_BOUNDS)


def _lane_reduce(v, op, lane):
    for sh in (8, 4, 2, 1):
        v = op(v, _shuffle(v, lane ^ sh))
    return v


@functools.partial(
    pl.kernel,
    out_type=(
        jax.ShapeDtypeStruct((NW, L), jnp.float32),
        jax.ShapeDtypeStruct((NW, L), jnp.int32),
    ),
    mesh=_MESH,
    scratch_types=[
        pltpu.VMEM((CH,), jnp.float32),
        pltpu.VMEM((CH,), jnp.float32),
        pltpu.VMEM((CH,), jnp.float32),
        pltpu.VMEM((CH,), jnp.float32),
        pltpu.VMEM((L,), jnp.float32),
        pltpu.VMEM((L,), jnp.int32),
        pltpu.SemaphoreType.DMA,
        pltpu.SemaphoreType.DMA,
        pltpu.SemaphoreType.DMA,
        pltpu.SemaphoreType.DMA,
    ],
)
def _sc_scan(x_hbm, vals_hbm, idxs_hbm, buf0, buf1, buf2, buf3,
             stage_v, stage_i, sem0, sem1, sem2, sem3):
    wid = lax.axis_index("s") * NC + lax.axis_index("c")
    row0 = wid * RPW
    bufs = (buf0, buf1, buf2, buf3)
    sems = (sem0, sem1, sem2, sem3)
    lane = lax.iota(jnp.int32, L)

    def start_dyn(c, b):
        row = row0 + c // CPR
        col = pl.multiple_of((c % CPR) * CH, CH)
        pltpu.async_copy(x_hbm.at[row, pl.ds(col, CH)], bufs[b], sems[b])

    def scan_chunk(buf):
        init = tuple(jnp.full((L,), NEG, jnp.float32)
                     for _ in range(UNROLL))

        @plsc.parallel_loop(0, VPC, step=UNROLL, unroll=2, carry=init)
        def accs(i, accs_in):
            return tuple(
                jnp.maximum(accs_in[u], buf[pl.ds((i + u) * L, L)])
                for u in range(UNROLL)
            )

        acc = accs[0]
        for u in range(1, UNROLL):
            acc = jnp.maximum(acc, accs[u])
        return acc

    def chunk_step(c, b, cmv):
        pltpu.make_async_copy(x_hbm.at[0, pl.ds(0, CH)], bufs[b],
                              sems[b]).wait()
        m_c = _lane_reduce(scan_chunk(bufs[b]), jnp.maximum, lane)

        @pl.when(c + NBUF < NCHUNK)
        def _():
            start_dyn(c + NBUF, b)

        return jnp.where(lane == c, m_c, cmv)

    for b in range(NBUF):
        start_dyn(b, b)

    @pl.loop(0, NCHUNK, step=NBUF,
             init_carry=jnp.full((L,), NEG, jnp.float32))
    def cmv(c4, cmv_in):
        for b in range(NBUF):
            cmv_in = chunk_step(c4 + b, b, cmv_in)
        return cmv_in

    wmax = _lane_reduce(cmv, jnp.maximum, lane)
    cid_v = _lane_reduce(jnp.where(cmv == wmax, lane, BIG), jnp.minimum,
                         lane)
    cid = cid_v[0]

    rrow = row0 + cid // CPR
    rcol = pl.multiple_of((cid % CPR) * CH, CH)
    pltpu.async_copy(x_hbm.at[rrow, pl.ds(rcol, CH)], buf0, sem0).wait()

    def rbody(i, rixs):
        return tuple(
            jnp.minimum(rixs[u],
                        jnp.where(buf0[pl.ds((i * RU + u) * L, L)] == wmax,
                                  i, BIG))
            for u in range(RU)
        )

    rixs = lax.fori_loop(0, VPC // RU, rbody,
                         tuple(jnp.full((L,), BIG, jnp.int32)
                               for _ in range(RU)))
    fl = jnp.full((L,), BIG, jnp.int32)
    for u in range(RU):
        f = rrow * COLS + rcol + (rixs[u] * RU + u) * L + lane
        fl = jnp.minimum(fl, jnp.where(rixs[u] == BIG, BIG, f))
    widx = _lane_reduce(fl, jnp.minimum, lane)

    stage_v[...] = wmax
    stage_i[...] = widx
    pltpu.sync_copy(stage_v, vals_hbm.at[wid])
    pltpu.sync_copy(stage_i, idxs_hbm.at[wid])


def _merge_body(vals_ref, idxs_ref, out_ref):
    def body(w, carry):
        bv, bi = carry
        v = vals_ref[w, 0]
        ix = idxs_ref[w, 0]
        better = (v > bv) | ((v == bv) & (ix < bi))
        return (jnp.where(better, v, bv), jnp.where(better, ix, bi))

    _, bi = lax.fori_loop(0, NW, body,
                          (jnp.float32(NEG), jnp.int32(BIG)))
    out_ref[0] = bi


def kernel(x):
    vals, idxs = _sc_scan(x)
    merged = pl.pallas_call(
        _merge_body,
        in_specs=[
            pl.BlockSpec(memory_space=pltpu.SMEM),
            pl.BlockSpec(memory_space=pltpu.SMEM),
        ],
        out_specs=pl.BlockSpec(memory_space=pltpu.SMEM),
        out_shape=jax.ShapeDtypeStruct((1,), jnp.int32),
    )(vals, idxs)
    return merged[0].astype(jnp.int64)


# speedup vs baseline: 1.7874x; 1.0635x over previous
import functools

import jax
import jax.numpy as jnp
from jax import lax
from jax.experimental import pallas as pl
from jax.experimental.pallas import tpu as pltpu
from jax.experimental.pallas import tpu_sc as plsc

NC = 2
NS = 16
L = 16
NW = NC * NS
ROWS = 128
COLS = 32768
SC_ROWS = 64
TC_ROWS = ROWS - SC_ROWS
TCBLK = 8192
SLAB = SC_ROWS * COLS // NW
CH = 8192
NCHUNK = SLAB // CH
VPC = CH // L
U = 4
BIG = 2**31 - 1
NEG = float("-inf")

_MESH = plsc.VectorSubcoreMesh(core_axis_name="c", subcore_axis_name="s",
                               num_cores=NC, num_subcores=NS)

_GDN = lax.GatherDimensionNumbers(
    offset_dims=(), collapsed_slice_dims=(0,), start_index_map=(0,))


def _shuffle(v, idx):
    return lax.gather(v, idx[:, None], _GDN, (1,),
                      mode=lax.GatherScatterMode.PROMISE_IN_BOUNDS)


@functools.partial(
    pl.kernel,
    out_type=(
        jax.ShapeDtypeStruct((NW, L), jnp.float32),
        jax.ShapeDtypeStruct((NW, L), jnp.int32),
    ),
    mesh=_MESH,
    scratch_types=[
        pltpu.VMEM((CH,), jnp.float32),
        pltpu.VMEM((CH,), jnp.float32),
        pltpu.VMEM((CH,), jnp.float32),
        pltpu.VMEM((CH,), jnp.float32),
        pltpu.VMEM((CH,), jnp.float32),
        pltpu.VMEM((CH,), jnp.float32),
        pltpu.VMEM((CH,), jnp.float32),
        pltpu.VMEM((CH,), jnp.float32),
        pltpu.VMEM((L,), jnp.float32),
        pltpu.VMEM((L,), jnp.int32),
        pltpu.SemaphoreType.DMA,
        pltpu.SemaphoreType.DMA,
        pltpu.SemaphoreType.DMA,
        pltpu.SemaphoreType.DMA,
        pltpu.SemaphoreType.DMA,
        pltpu.SemaphoreType.DMA,
        pltpu.SemaphoreType.DMA,
        pltpu.SemaphoreType.DMA,
    ],
)
def _sc_scan(x_hbm, vals_hbm, idxs_hbm, buf0, buf1, buf2, buf3,
             buf4, buf5, buf6, buf7, stage_v, stage_i,
             sem0, sem1, sem2, sem3, sem4, sem5, sem6, sem7):
    wid = lax.axis_index("s") * NC + lax.axis_index("c")
    base = wid * SLAB
    bufs = (buf0, buf1, buf2, buf3, buf4, buf5, buf6, buf7)
    sems = (sem0, sem1, sem2, sem3, sem4, sem5, sem6, sem7)
    lane = lax.iota(jnp.int32, L)

    for c in range(NCHUNK):
        off = base + c * CH
        row = off // COLS
        col = pl.multiple_of(off % COLS, CH)
        pltpu.async_copy(x_hbm.at[row, pl.ds(col, CH)], bufs[c], sems[c])

    accs = tuple(jnp.full((L,), NEG, jnp.float32) for _ in range(U))
    poss = tuple(jnp.full((L,), 0, jnp.int32) for _ in range(U))
    for c in range(NCHUNK):
        pltpu.make_async_copy(x_hbm.at[0, pl.ds(0, CH)], bufs[c],
                              sems[c]).wait()
        buf = bufs[c]

        @plsc.parallel_loop(0, VPC, step=U, unroll=2, carry=(accs, poss))
        def res(i, carry, buf=buf, c=c):
            a, p = carry
            gvec = jnp.full((L,), i + c * VPC)
            na, np_ = [], []
            for u in range(U):
                v = buf[pl.ds((i + u) * L, L)]
                gt = v > a[u]
                na.append(jnp.where(gt, v, a[u]))
                np_.append(jnp.where(gt, gvec, p[u]))
            return (tuple(na), tuple(np_))

        accs, poss = res

    bv = accs[0]
    bi = base + (poss[0] + 0) * L + lane
    for u in range(1, U):
        fv = accs[u]
        fi = base + (poss[u] + u) * L + lane
        better = (fv > bv) | ((fv == bv) & (fi < bi))
        bv = jnp.where(better, fv, bv)
        bi = jnp.where(better, fi, bi)

    for sh in (8, 4, 2, 1):
        idx2 = lane ^ sh
        v2 = _shuffle(bv, idx2)
        i2 = _shuffle(bi, idx2)
        better = (v2 > bv) | ((v2 == bv) & (i2 < bi))
        bv = jnp.where(better, v2, bv)
        bi = jnp.where(better, i2, bi)

    stage_v[...] = bv
    stage_i[...] = bi
    pltpu.sync_copy(stage_v, vals_hbm.at[wid])
    pltpu.sync_copy(stage_i, idxs_hbm.at[wid])


def _tc_body(x_ref, val_ref, idx_ref, rmax_ref, ridx_ref):
    b = pl.program_id(0)

    @pl.when(b == 0)
    def _init():
        rmax_ref[0] = -jnp.inf
        ridx_ref[0] = jnp.int32(BIG)

    xb = x_ref[...]
    m = jnp.max(xb)

    @pl.when(m >= rmax_ref[0])
    def _update():
        rows = lax.broadcasted_iota(jnp.int32, (TC_ROWS, TCBLK), 0)
        cols = lax.broadcasted_iota(jnp.int32, (TC_ROWS, TCBLK), 1)
        flat = (rows + SC_ROWS) * COLS + (b * TCBLK + cols)
        cand = jnp.min(jnp.where(xb == m, flat, jnp.int32(BIG)))
        old_m = rmax_ref[0]
        old_i = ridx_ref[0]
        better = (m > old_m) | (cand < old_i)
        ridx_ref[0] = jnp.where(better, cand, old_i)
        rmax_ref[0] = jnp.where(m > old_m, m, old_m)

    @pl.when(b == pl.num_programs(0) - 1)
    def _fin():
        val_ref[0] = rmax_ref[0]
        idx_ref[0] = ridx_ref[0]


def _merge_body(vals_ref, idxs_ref, tcv_ref, tci_ref, out_ref):
    v = vals_ref[...]
    ix = idxs_ref[...]
    m = jnp.max(v)
    cand = jnp.min(jnp.where(v == m, ix, jnp.int32(BIG)))
    tv = tcv_ref[0]
    ti = tci_ref[0]
    better_tc = (tv > m) | ((tv == m) & (ti < cand))
    out_ref[0] = jnp.where(better_tc, ti, cand)


def kernel(x):
    vals, idxs = _sc_scan(x)
    tcv, tci = pl.pallas_call(
        _tc_body,
        grid=(COLS // TCBLK,),
        in_specs=[pl.BlockSpec((TC_ROWS, TCBLK), lambda b: (1, b))],
        out_specs=(
            pl.BlockSpec(memory_space=pltpu.SMEM),
            pl.BlockSpec(memory_space=pltpu.SMEM),
        ),
        out_shape=(
            jax.ShapeDtypeStruct((1,), jnp.float32),
            jax.ShapeDtypeStruct((1,), jnp.int32),
        ),
        scratch_shapes=[
            pltpu.SMEM((1,), jnp.float32),
            pltpu.SMEM((1,), jnp.int32),
        ],
    )(x)
    merged = pl.pallas_call(
        _merge_body,
        in_specs=[
            pl.BlockSpec((NW, L), lambda: (0, 0)),
            pl.BlockSpec((NW, L), lambda: (0, 0)),
            pl.BlockSpec(memory_space=pltpu.SMEM),
            pl.BlockSpec(memory_space=pltpu.SMEM),
        ],
        out_specs=pl.BlockSpec(memory_space=pltpu.SMEM),
        out_shape=jax.ShapeDtypeStruct((1,), jnp.int32),
    )(vals, idxs, tcv, tci)
    return merged[0].astype(jnp.int64)


# speedup vs baseline: 1.8523x; 1.0364x over previous
import functools

import jax
import jax.numpy as jnp
from jax import lax
from jax.experimental import pallas as pl
from jax.experimental.pallas import tpu as pltpu
from jax.experimental.pallas import tpu_sc as plsc

NC = 2
NS = 16
L = 16
NW = NC * NS
ROWS = 128
COLS = 32768
SC_ROWS = 64
TC_ROWS = ROWS - SC_ROWS
TCRB = 16
SLAB = SC_ROWS * COLS // NW
CH = 8192
NCHUNK = SLAB // CH
VPC = CH // L
U = 4
BIG = 2**31 - 1
NEG = float("-inf")

_MESH = plsc.VectorSubcoreMesh(core_axis_name="c", subcore_axis_name="s",
                               num_cores=NC, num_subcores=NS)

_GDN = lax.GatherDimensionNumbers(
    offset_dims=(), collapsed_slice_dims=(0,), start_index_map=(0,))


def _shuffle(v, idx):
    return lax.gather(v, idx[:, None], _GDN, (1,),
                      mode=lax.GatherScatterMode.PROMISE_IN_BOUNDS)


@functools.partial(
    pl.kernel,
    out_type=(
        jax.ShapeDtypeStruct((NW, L), jnp.float32),
        jax.ShapeDtypeStruct((NW, L), jnp.int32),
    ),
    mesh=_MESH,
    scratch_types=[
        pltpu.VMEM((CH,), jnp.float32),
        pltpu.VMEM((CH,), jnp.float32),
        pltpu.VMEM((CH,), jnp.float32),
        pltpu.VMEM((CH,), jnp.float32),
        pltpu.VMEM((CH,), jnp.float32),
        pltpu.VMEM((CH,), jnp.float32),
        pltpu.VMEM((CH,), jnp.float32),
        pltpu.VMEM((CH,), jnp.float32),
        pltpu.VMEM((L,), jnp.float32),
        pltpu.VMEM((L,), jnp.int32),
        pltpu.SemaphoreType.DMA,
        pltpu.SemaphoreType.DMA,
        pltpu.SemaphoreType.DMA,
        pltpu.SemaphoreType.DMA,
        pltpu.SemaphoreType.DMA,
        pltpu.SemaphoreType.DMA,
        pltpu.SemaphoreType.DMA,
        pltpu.SemaphoreType.DMA,
    ],
)
def _sc_scan(x_hbm, vals_hbm, idxs_hbm, buf0, buf1, buf2, buf3,
             buf4, buf5, buf6, buf7, stage_v, stage_i,
             sem0, sem1, sem2, sem3, sem4, sem5, sem6, sem7):
    wid = lax.axis_index("s") * NC + lax.axis_index("c")
    base = wid * SLAB
    bufs = (buf0, buf1, buf2, buf3, buf4, buf5, buf6, buf7)
    sems = (sem0, sem1, sem2, sem3, sem4, sem5, sem6, sem7)
    lane = lax.iota(jnp.int32, L)

    for c in range(NCHUNK):
        off = base + c * CH
        row = off // COLS
        col = pl.multiple_of(off % COLS, CH)
        pltpu.async_copy(x_hbm.at[row, pl.ds(col, CH)], bufs[c], sems[c])

    accs = tuple(jnp.full((L,), NEG, jnp.float32) for _ in range(U))
    poss = tuple(jnp.full((L,), 0, jnp.int32) for _ in range(U))
    for c in range(NCHUNK):
        pltpu.make_async_copy(x_hbm.at[0, pl.ds(0, CH)], bufs[c],
                              sems[c]).wait()
        buf = bufs[c]

        @plsc.parallel_loop(0, VPC, step=U, unroll=2, carry=(accs, poss))
        def res(i, carry, buf=buf, c=c):
            a, p = carry
            gvec = jnp.full((L,), i + c * VPC)
            na, np_ = [], []
            for u in range(U):
                v = buf[pl.ds((i + u) * L, L)]
                gt = v > a[u]
                na.append(jnp.where(gt, v, a[u]))
                np_.append(jnp.where(gt, gvec, p[u]))
            return (tuple(na), tuple(np_))

        accs, poss = res

    bv = accs[0]
    bi = base + (poss[0] + 0) * L + lane
    for u in range(1, U):
        fv = accs[u]
        fi = base + (poss[u] + u) * L + lane
        better = (fv > bv) | ((fv == bv) & (fi < bi))
        bv = jnp.where(better, fv, bv)
        bi = jnp.where(better, fi, bi)

    for sh in (8, 4, 2, 1):
        idx2 = lane ^ sh
        v2 = _shuffle(bv, idx2)
        i2 = _shuffle(bi, idx2)
        better = (v2 > bv) | ((v2 == bv) & (i2 < bi))
        bv = jnp.where(better, v2, bv)
        bi = jnp.where(better, i2, bi)

    stage_v[...] = bv
    stage_i[...] = bi
    pltpu.sync_copy(stage_v, vals_hbm.at[wid])
    pltpu.sync_copy(stage_i, idxs_hbm.at[wid])


def _tc_body(x_ref, val_ref, idx_ref, rmax_ref, ridx_ref):
    b = pl.program_id(0)

    @pl.when(b == 0)
    def _init():
        rmax_ref[0] = -jnp.inf
        ridx_ref[0] = jnp.int32(BIG)

    xb = x_ref[...]
    m = jnp.max(xb)

    @pl.when(m >= rmax_ref[0])
    def _update():
        rows = lax.broadcasted_iota(jnp.int32, (TCRB, COLS), 0)
        cols = lax.broadcasted_iota(jnp.int32, (TCRB, COLS), 1)
        flat = (rows + SC_ROWS + b * TCRB) * COLS + cols
        cand = jnp.min(jnp.where(xb == m, flat, jnp.int32(BIG)))
        old_m = rmax_ref[0]
        old_i = ridx_ref[0]
        better = (m > old_m) | (cand < old_i)
        ridx_ref[0] = jnp.where(better, cand, old_i)
        rmax_ref[0] = jnp.where(m > old_m, m, old_m)

    @pl.when(b == pl.num_programs(0) - 1)
    def _fin():
        val_ref[0] = rmax_ref[0]
        idx_ref[0] = ridx_ref[0]


def _merge_body(vals_ref, idxs_ref, tcv_ref, tci_ref, out_ref):
    v = vals_ref[...]
    ix = idxs_ref[...]
    m = jnp.max(v)
    cand = jnp.min(jnp.where(v == m, ix, jnp.int32(BIG)))
    tv = tcv_ref[0]
    ti = tci_ref[0]
    better_tc = (tv > m) | ((tv == m) & (ti < cand))
    out_ref[0] = jnp.where(better_tc, ti, cand)


def kernel(x):
    vals, idxs = _sc_scan(x)
    tcv, tci = pl.pallas_call(
        _tc_body,
        grid=(TC_ROWS // TCRB,),
        in_specs=[pl.BlockSpec((TCRB, COLS),
                               lambda b: (b + SC_ROWS // TCRB, 0))],
        out_specs=(
            pl.BlockSpec(memory_space=pltpu.SMEM),
            pl.BlockSpec(memory_space=pltpu.SMEM),
        ),
        out_shape=(
            jax.ShapeDtypeStruct((1,), jnp.float32),
            jax.ShapeDtypeStruct((1,), jnp.int32),
        ),
        scratch_shapes=[
            pltpu.SMEM((1,), jnp.float32),
            pltpu.SMEM((1,), jnp.int32),
        ],
    )(x)
    merged = pl.pallas_call(
        _merge_body,
        in_specs=[
            pl.BlockSpec((NW, L), lambda: (0, 0)),
            pl.BlockSpec((NW, L), lambda: (0, 0)),
            pl.BlockSpec(memory_space=pltpu.SMEM),
            pl.BlockSpec(memory_space=pltpu.SMEM),
        ],
        out_specs=pl.BlockSpec(memory_space=pltpu.SMEM),
        out_shape=jax.ShapeDtypeStruct((1,), jnp.int32),
    )(vals, idxs, tcv, tci)
    return merged[0].astype(jnp.int64)


# speedup vs baseline: 1.8754x; 1.0125x over previous
import functools

import jax
import jax.numpy as jnp
from jax import lax
from jax.experimental import pallas as pl
from jax.experimental.pallas import tpu as pltpu
from jax.experimental.pallas import tpu_sc as plsc

NC = 2
NS = 16
L = 16
NW = NC * NS
ROWS = 128
COLS = 32768
SC_ROWS = 48
TC_ROWS = ROWS - SC_ROWS
TCRB = 16
SLAB = SC_ROWS * COLS // NW
CH = 8192
NCHUNK = SLAB // CH
VPC = CH // L
U = 4
BIG = 2**31 - 1
NEG = float("-inf")

_MESH = plsc.VectorSubcoreMesh(core_axis_name="c", subcore_axis_name="s",
                               num_cores=NC, num_subcores=NS)

_GDN = lax.GatherDimensionNumbers(
    offset_dims=(), collapsed_slice_dims=(0,), start_index_map=(0,))


def _shuffle(v, idx):
    return lax.gather(v, idx[:, None], _GDN, (1,),
                      mode=lax.GatherScatterMode.PROMISE_IN_BOUNDS)


@functools.partial(
    pl.kernel,
    out_type=(
        jax.ShapeDtypeStruct((NW, L), jnp.float32),
        jax.ShapeDtypeStruct((NW, L), jnp.int32),
    ),
    mesh=_MESH,
    scratch_types=[
        pltpu.VMEM((CH,), jnp.float32),
        pltpu.VMEM((CH,), jnp.float32),
        pltpu.VMEM((CH,), jnp.float32),
        pltpu.VMEM((CH,), jnp.float32),
        pltpu.VMEM((CH,), jnp.float32),
        pltpu.VMEM((CH,), jnp.float32),
        pltpu.VMEM((CH,), jnp.float32),
        pltpu.VMEM((CH,), jnp.float32),
        pltpu.VMEM((L,), jnp.float32),
        pltpu.VMEM((L,), jnp.int32),
        pltpu.SemaphoreType.DMA,
        pltpu.SemaphoreType.DMA,
        pltpu.SemaphoreType.DMA,
        pltpu.SemaphoreType.DMA,
        pltpu.SemaphoreType.DMA,
        pltpu.SemaphoreType.DMA,
        pltpu.SemaphoreType.DMA,
        pltpu.SemaphoreType.DMA,
    ],
)
def _sc_scan(x_hbm, vals_hbm, idxs_hbm, buf0, buf1, buf2, buf3,
             buf4, buf5, buf6, buf7, stage_v, stage_i,
             sem0, sem1, sem2, sem3, sem4, sem5, sem6, sem7):
    wid = lax.axis_index("s") * NC + lax.axis_index("c")
    base = wid * SLAB
    bufs = (buf0, buf1, buf2, buf3, buf4, buf5, buf6, buf7)
    sems = (sem0, sem1, sem2, sem3, sem4, sem5, sem6, sem7)
    lane = lax.iota(jnp.int32, L)

    for c in range(NCHUNK):
        off = base + c * CH
        row = off // COLS
        col = pl.multiple_of(off % COLS, CH)
        pltpu.async_copy(x_hbm.at[row, pl.ds(col, CH)], bufs[c], sems[c])

    accs = tuple(jnp.full((L,), NEG, jnp.float32) for _ in range(U))
    poss = tuple(jnp.full((L,), 0, jnp.int32) for _ in range(U))
    for c in range(NCHUNK):
        pltpu.make_async_copy(x_hbm.at[0, pl.ds(0, CH)], bufs[c],
                              sems[c]).wait()
        buf = bufs[c]

        @plsc.parallel_loop(0, VPC, step=U, unroll=2, carry=(accs, poss))
        def res(i, carry, buf=buf, c=c):
            a, p = carry
            gvec = jnp.full((L,), i + c * VPC)
            na, np_ = [], []
            for u in range(U):
                v = buf[pl.ds((i + u) * L, L)]
                gt = v > a[u]
                na.append(jnp.where(gt, v, a[u]))
                np_.append(jnp.where(gt, gvec, p[u]))
            return (tuple(na), tuple(np_))

        accs, poss = res

    bv = accs[0]
    bi = base + (poss[0] + 0) * L + lane
    for u in range(1, U):
        fv = accs[u]
        fi = base + (poss[u] + u) * L + lane
        better = (fv > bv) | ((fv == bv) & (fi < bi))
        bv = jnp.where(better, fv, bv)
        bi = jnp.where(better, fi, bi)

    for sh in (8, 4, 2, 1):
        idx2 = lane ^ sh
        v2 = _shuffle(bv, idx2)
        i2 = _shuffle(bi, idx2)
        better = (v2 > bv) | ((v2 == bv) & (i2 < bi))
        bv = jnp.where(better, v2, bv)
        bi = jnp.where(better, i2, bi)

    stage_v[...] = bv
    stage_i[...] = bi
    pltpu.sync_copy(stage_v, vals_hbm.at[wid])
    pltpu.sync_copy(stage_i, idxs_hbm.at[wid])


def _tc_body(x_ref, val_ref, idx_ref, rmax_ref, ridx_ref):
    b = pl.program_id(0)

    @pl.when(b == 0)
    def _init():
        rmax_ref[0] = -jnp.inf
        ridx_ref[0] = jnp.int32(BIG)

    xb = x_ref[...]
    m = jnp.max(xb)

    @pl.when(m >= rmax_ref[0])
    def _update():
        rows = lax.broadcasted_iota(jnp.int32, (TCRB, COLS), 0)
        cols = lax.broadcasted_iota(jnp.int32, (TCRB, COLS), 1)
        flat = (rows + SC_ROWS + b * TCRB) * COLS + cols
        cand = jnp.min(jnp.where(xb == m, flat, jnp.int32(BIG)))
        old_m = rmax_ref[0]
        old_i = ridx_ref[0]
        better = (m > old_m) | (cand < old_i)
        ridx_ref[0] = jnp.where(better, cand, old_i)
        rmax_ref[0] = jnp.where(m > old_m, m, old_m)

    @pl.when(b == pl.num_programs(0) - 1)
    def _fin():
        val_ref[0] = rmax_ref[0]
        idx_ref[0] = ridx_ref[0]


def _merge_body(vals_ref, idxs_ref, tcv_ref, tci_ref, out_ref):
    v = vals_ref[...]
    ix = idxs_ref[...]
    m = jnp.max(v)
    cand = jnp.min(jnp.where(v == m, ix, jnp.int32(BIG)))
    tv = tcv_ref[0]
    ti = tci_ref[0]
    better_tc = (tv > m) | ((tv == m) & (ti < cand))
    out_ref[0] = jnp.where(better_tc, ti, cand)


def kernel(x):
    vals, idxs = _sc_scan(x)
    tcv, tci = pl.pallas_call(
        _tc_body,
        grid=(TC_ROWS // TCRB,),
        in_specs=[pl.BlockSpec((TCRB, COLS),
                               lambda b: (b + SC_ROWS // TCRB, 0))],
        out_specs=(
            pl.BlockSpec(memory_space=pltpu.SMEM),
            pl.BlockSpec(memory_space=pltpu.SMEM),
        ),
        out_shape=(
            jax.ShapeDtypeStruct((1,), jnp.float32),
            jax.ShapeDtypeStruct((1,), jnp.int32),
        ),
        scratch_shapes=[
            pltpu.SMEM((1,), jnp.float32),
            pltpu.SMEM((1,), jnp.int32),
        ],
    )(x)
    merged = pl.pallas_call(
        _merge_body,
        in_specs=[
            pl.BlockSpec((NW, L), lambda: (0, 0)),
            pl.BlockSpec((NW, L), lambda: (0, 0)),
            pl.BlockSpec(memory_space=pltpu.SMEM),
            pl.BlockSpec(memory_space=pltpu.SMEM),
        ],
        out_specs=pl.BlockSpec(memory_space=pltpu.SMEM),
        out_shape=jax.ShapeDtypeStruct((1,), jnp.int32),
    )(vals, idxs, tcv, tci)
    return merged[0].astype(jnp.int64)
